# baseline (speedup 1.0000x reference)
import functools

import jax
import jax.numpy as jnp
from jax import lax
from jax.experimental import pallas as pl
from jax.experimental.pallas import tpu as pltpu
from jax.experimental.pallas import tpu_sc as plsc

N = 10000
D = 128
E = 320000
NW = 32
K = 128
NB = 8
BPW = 80
PER_W = BPW * K
EPAD = NW * PER_W
NBLK = BPW // NB
BPW0 = 160
BPW1 = 0
NBLK0 = BPW0 // NB
NBLK1 = BPW1 // NB
NPAD = 10240
SL = NPAD // 16
NRING = 2
LEAD = 1

_MIN_NORM = 1e-15
_BALL_EPS = 4e-3

_MESH = dict(core_axis_name="c", subcore_axis_name="s")


@functools.partial(
    pl.kernel,
    out_type=jax.ShapeDtypeStruct((2, NPAD), jnp.float32),
    mesh=plsc.VectorSubcoreMesh(**_MESH),
    scratch_types=[
        pltpu.VMEM((NB, K), jnp.int32),
        pltpu.VMEM((K,), jnp.float32),
        pltpu.VMEM((K,), jnp.float32),
        pltpu.VMEM_SHARED((NPAD,), jnp.float32),
        pltpu.SemaphoreType.DMA,
    ],
)
def _deg_sc(row_hbm, out_hbm, ridx8, ones_v, zero_v, acc_sh, sem1):
    c = lax.axis_index("c")
    s = lax.axis_index("s")
    wid = s * 2 + c
    for j in range(K // 16):
        ones_v[pl.ds(j * 16, 16)] = jnp.full((16,), 1.0, jnp.float32)
        zero_v[pl.ds(j * 16, 16)] = jnp.zeros((16,), jnp.float32)
    for j in range(SL // K):
        pltpu.sync_copy(zero_v, acc_sh.at[pl.ds(s * SL + j * K, K)])
    plsc.subcore_barrier()

    def blk(b, carry):
        boff = wid * BPW + b * NB
        pltpu.sync_copy(row_hbm.at[pl.ds(boff, NB)], ridx8)
        descs = []
        for j in range(NB):
            descs.append(
                pltpu.async_copy(ones_v, acc_sh.at[ridx8.at[j]], sem1, add=True))
        for d in descs:
            d.wait()
        return carry

    lax.fori_loop(0, NBLK, blk, 0)
    plsc.subcore_barrier()
    pltpu.sync_copy(acc_sh.at[pl.ds(s * SL, SL)], out_hbm.at[c, pl.ds(s * SL, SL)])


@functools.partial(
    pl.kernel,
    out_type=[
        jax.ShapeDtypeStruct((2, NPAD, D), jnp.float32),
        jax.ShapeDtypeStruct((2, NPAD), jnp.float32),
    ],
    mesh=plsc.VectorSubcoreMesh(**_MESH),
    scratch_types=[
        pltpu.VMEM((2, NB, K), jnp.int32),
        pltpu.VMEM((2, NB, K), jnp.int32),
        pltpu.VMEM((NRING, K, D), jnp.float32),
        pltpu.VMEM((NRING, K), jnp.float32),
        pltpu.VMEM_SHARED((NPAD, D), jnp.float32),
        pltpu.VMEM_SHARED((NPAD,), jnp.float32),
        [pltpu.SemaphoreType.DMA] * 2,
        [pltpu.SemaphoreType.DMA] * 2,
        [pltpu.SemaphoreType.DMA] * NRING,
        [pltpu.SemaphoreType.DMA] * NRING,
        [pltpu.SemaphoreType.DMA] * NRING,
        [pltpu.SemaphoreType.DMA] * NRING,
    ],
)
def _agg_sc(row_hbm, col_hbm, t_hbm, f_hbm, outv_hbm, outf_hbm,
            cidx2, ridx2, rows_r, fr, accv, accf,
            semr, semc, semg, sems, semfg, semfs):
    c = lax.axis_index("c")
    s = lax.axis_index("s")

    def zbody(i, carry):
        for j in range(D // 16):
            rows_r[0, i, pl.ds(j * 16, 16)] = jnp.zeros((16,), jnp.float32)
        return carry

    lax.fori_loop(0, K, zbody, 0)
    for j in range(K // 16):
        fr[0, pl.ds(j * 16, 16)] = jnp.zeros((16,), jnp.float32)
    for j in range(SL // K):
        pltpu.sync_copy(rows_r.at[0], accv.at[pl.ds(s * SL + j * K, K)])
        pltpu.sync_copy(fr.at[0], accf.at[pl.ds(s * SL + j * K, K)])
    plsc.subcore_barrier()

    base = jnp.where(c == 0, s * BPW0, jnp.minimum(16 * BPW0 + s * BPW1,
                                                   EPAD // K - NB))
    nblk = jnp.where(c == 0, NBLK0, NBLK1)
    pltpu.async_copy(row_hbm.at[pl.ds(base, NB)], ridx2.at[0], semr[0]).wait()
    pltpu.async_copy(col_hbm.at[pl.ds(base, NB)], cidx2.at[0], semc[0]).wait()

    def one_block(noff, cur, nxt):
        ir = pltpu.async_copy(row_hbm.at[pl.ds(noff, NB)], ridx2.at[nxt],
                              semr[nxt])
        ic = pltpu.async_copy(col_hbm.at[pl.ds(noff, NB)], cidx2.at[nxt],
                              semc[nxt])
        ridx8 = ridx2.at[cur]
        cidx8 = cidx2.at[cur]
        gr = [None] * NRING
        sr = [None] * NRING
        gf = [None] * NRING
        sf = [None] * NRING
        for j in range(NB + LEAD):
            if j < NB:
                q = j % NRING
                if sr[q] is not None:
                    sr[q].wait()
                    sf[q].wait()
                gr[q] = pltpu.async_copy(t_hbm.at[cidx8.at[j]], rows_r.at[q],
                                         semg[q])
                gf[q] = pltpu.async_copy(f_hbm.at[cidx8.at[j]], fr.at[q],
                                         semfg[q])
            if j >= LEAD:
                p = (j - LEAD) % NRING
                gr[p].wait()
                gf[p].wait()
                sr[p] = pltpu.async_copy(rows_r.at[p],
                                         accv.at[ridx8.at[j - LEAD]],
                                         sems[p], add=True)
                sf[p] = pltpu.async_copy(fr.at[p],
                                         accf.at[ridx8.at[j - LEAD]],
                                         semfs[p], add=True)
        for q in range(NRING):
            if sr[q] is not None:
                sr[q].wait()
                sf[q].wait()
        ir.wait()
        ic.wait()

    def blk2(i, carry):
        b0 = i * 2
        one_block(base + ((b0 + 1) % nblk) * NB, 0, 1)
        one_block(base + ((b0 + 2) % nblk) * NB, 1, 0)
        return carry

    lax.fori_loop(0, nblk // 2, blk2, 0)
    plsc.subcore_barrier()
    pltpu.sync_copy(accv.at[pl.ds(s * SL, SL)], outv_hbm.at[c, pl.ds(s * SL, SL)])
    pltpu.sync_copy(accf.at[pl.ds(s * SL, SL)], outf_hbm.at[c, pl.ds(s * SL, SL)])


def _norm_kd(x):
    return jnp.clip(jnp.sqrt(jnp.sum(x * x, axis=-1, keepdims=True)),
                    _MIN_NORM, None)


def _proj(x, c):
    n = _norm_kd(x)
    maxnorm = (1.0 - _BALL_EPS) / jnp.sqrt(c)
    return jnp.where(n > maxnorm, x / n * maxnorm, x)


def _expmap0(u, c):
    sqrt_c = jnp.sqrt(c)
    u_norm = _norm_kd(u)
    return jnp.tanh(sqrt_c * u_norm) * u / (sqrt_c * u_norm)


def _mobius_add(x, y, c):
    x2 = jnp.sum(x * x, axis=-1, keepdims=True)
    y2 = jnp.sum(y * y, axis=-1, keepdims=True)
    xy = jnp.sum(x * y, axis=-1, keepdims=True)
    num = (1.0 + 2.0 * c * xy + c * y2) * x + (1.0 - c * x2) * y
    denom = 1.0 + 2.0 * c * xy + (c ** 2) * x2 * y2
    return num / jnp.clip(denom, _MIN_NORM, None)


def _p2k(x, c):
    return 2.0 * x / (1.0 + c * jnp.sum(x * x, axis=-1, keepdims=True))


def _k2p(x, c):
    denom = 1.0 + jnp.sqrt(
        jnp.clip(1.0 - c * jnp.sum(x * x, axis=-1, keepdims=True),
                 _MIN_NORM, None))
    return x / denom


def _dense_body(x_ref, w_ref, b_ref, d0_ref, d1_ref, c_ref, t_ref, f_ref):
    cc = c_ref[0, 0]
    xw = lax.dot_general(x_ref[...], w_ref[...], (((1,), (1,)), ((), ())),
                         preferred_element_type=jnp.float32)
    x_ = _proj(xw, cc)
    hb = _expmap0(b_ref[...], cc)
    x_ = _proj(_mobius_add(x_, hb, cc), cc)
    s = _p2k(x_, cc)
    lamb = lax.rsqrt(jnp.clip(1.0 - jnp.sum(s * s, axis=-1, keepdims=True),
                              _MIN_NORM, None))
    deg = 1.0 + d0_ref[...] + d1_ref[...]
    f = lax.rsqrt(deg) * lamb
    t_ref[...] = f * s
    f_ref[...] = f


def _fin_body(v_ref, s_ref, t_ref, f_ref, c_ref, o_ref):
    cc = c_ref[0, 0]
    num = v_ref[0] + v_ref[1] + t_ref[...]
    den = s_ref[0] + s_ref[1] + f_ref[...]
    out = _k2p(num / den, cc)
    o_ref[...] = jnp.where(out > 0, out, 0.01 * out)


_RB = 400


def _dense_call(x, W, bias, d0, d1, c_arr):
    return pl.pallas_call(
        _dense_body,
        grid=(N // _RB,),
        in_specs=[
            pl.BlockSpec((_RB, D), lambda i: (i, 0)),
            pl.BlockSpec((D, D), lambda i: (0, 0)),
            pl.BlockSpec((1, D), lambda i: (0, 0)),
            pl.BlockSpec((_RB, 1), lambda i: (i, 0)),
            pl.BlockSpec((_RB, 1), lambda i: (i, 0)),
            pl.BlockSpec((1, 1), lambda i: (0, 0)),
        ],
        out_specs=[
            pl.BlockSpec((_RB, D), lambda i: (i, 0)),
            pl.BlockSpec((_RB, 1), lambda i: (i, 0)),
        ],
        out_shape=[
            jax.ShapeDtypeStruct((N, D), jnp.float32),
            jax.ShapeDtypeStruct((N, 1), jnp.float32),
        ],
    )(x, W, bias, d0, d1, c_arr)


def _fin_call(aggv, aggs2, t, f, c_arr):
    return pl.pallas_call(
        _fin_body,
        grid=(N // _RB,),
        in_specs=[
            pl.BlockSpec((2, _RB, D), lambda i: (0, i, 0)),
            pl.BlockSpec((2, _RB, 1), lambda i: (0, i, 0)),
            pl.BlockSpec((_RB, D), lambda i: (i, 0)),
            pl.BlockSpec((_RB, 1), lambda i: (i, 0)),
            pl.BlockSpec((1, 1), lambda i: (0, 0)),
        ],
        out_specs=pl.BlockSpec((_RB, D), lambda i: (i, 0)),
        out_shape=jax.ShapeDtypeStruct((N, D), jnp.float32),
    )(aggv, aggs2, t, f, c_arr)


def kernel(x, edge_index, c_, W, bias):
    row = edge_index[0]
    col = edge_index[1]
    npd = EPAD - E
    pad_rows = N + jnp.arange(npd, dtype=jnp.int32) % (NPAD - N)
    rowp = jnp.concatenate([row, pad_rows])
    colp = jnp.concatenate([col, jnp.zeros((npd,), jnp.int32)])
    row3 = rowp.reshape(EPAD // K, K)
    col3 = colp.reshape(EPAD // K, K)

    degp = _deg_sc(row3)
    c_arr = jnp.full((1, 1), c_, jnp.float32)
    d0 = degp[0][:, None]
    d1 = degp[1][:, None]
    t, f = _dense_call(x, W, bias, d0, d1, c_arr)

    aggv, aggs = _agg_sc(row3, col3, t, f.reshape(N))
    aggs2 = aggs[:, :, None]
    return _fin_call(aggv, aggs2, t, f, c_arr)


# speedup vs baseline: 1.2487x; 1.2487x over previous
import functools

import jax
import jax.numpy as jnp
from jax import lax
from jax.experimental import pallas as pl
from jax.experimental.pallas import tpu as pltpu
from jax.experimental.pallas import tpu_sc as plsc

N = 10000
D = 128
E = 320000
NW = 32
K = 128
NB = 8
BPW = 80
PER_W = BPW * K
EPAD = NW * PER_W
NBLK = BPW // NB
BPW0 = 112
BPW1 = 48
NBLK0 = BPW0 // NB
NBLK1 = BPW1 // NB
NPAD = 10240
SL = NPAD // 16
NRING = 2
LEAD = 1

_MIN_NORM = 1e-15
_BALL_EPS = 4e-3

_MESH = dict(core_axis_name="c", subcore_axis_name="s")


@functools.partial(
    pl.kernel,
    out_type=jax.ShapeDtypeStruct((2, NPAD), jnp.float32),
    mesh=plsc.VectorSubcoreMesh(**_MESH),
    scratch_types=[
        pltpu.VMEM((NB, K), jnp.int32),
        pltpu.VMEM((K,), jnp.float32),
        pltpu.VMEM((K,), jnp.float32),
        pltpu.VMEM_SHARED((NPAD,), jnp.float32),
        pltpu.SemaphoreType.DMA,
    ],
)
def _deg_sc(row_hbm, out_hbm, ridx8, ones_v, zero_v, acc_sh, sem1):
    c = lax.axis_index("c")
    s = lax.axis_index("s")
    wid = s * 2 + c
    for j in range(K // 16):
        ones_v[pl.ds(j * 16, 16)] = jnp.full((16,), 1.0, jnp.float32)
        zero_v[pl.ds(j * 16, 16)] = jnp.zeros((16,), jnp.float32)
    for j in range(SL // K):
        pltpu.sync_copy(zero_v, acc_sh.at[pl.ds(s * SL + j * K, K)])
    plsc.subcore_barrier()

    def blk(b, carry):
        boff = wid * BPW + b * NB
        pltpu.sync_copy(row_hbm.at[pl.ds(boff, NB)], ridx8)
        descs = []
        for j in range(NB):
            descs.append(
                pltpu.async_copy(ones_v, acc_sh.at[ridx8.at[j]], sem1, add=True))
        for d in descs:
            d.wait()
        return carry

    lax.fori_loop(0, NBLK, blk, 0)
    plsc.subcore_barrier()
    pltpu.sync_copy(acc_sh.at[pl.ds(s * SL, SL)], out_hbm.at[c, pl.ds(s * SL, SL)])


@functools.partial(
    pl.kernel,
    out_type=[
        jax.ShapeDtypeStruct((2, NPAD, D), jnp.float32),
        jax.ShapeDtypeStruct((2, NPAD), jnp.float32),
    ],
    mesh=plsc.VectorSubcoreMesh(**_MESH),
    scratch_types=[
        pltpu.VMEM((2, NB, K), jnp.int32),
        pltpu.VMEM((2, NB, K), jnp.int32),
        pltpu.VMEM((NRING, K, D), jnp.float32),
        pltpu.VMEM((NRING, K), jnp.float32),
        pltpu.VMEM_SHARED((NPAD, D), jnp.float32),
        pltpu.VMEM_SHARED((NPAD,), jnp.float32),
        [pltpu.SemaphoreType.DMA] * 2,
        [pltpu.SemaphoreType.DMA] * 2,
        [pltpu.SemaphoreType.DMA] * NRING,
        [pltpu.SemaphoreType.DMA] * NRING,
        [pltpu.SemaphoreType.DMA] * NRING,
        [pltpu.SemaphoreType.DMA] * NRING,
    ],
)
def _agg_sc(row_hbm, col_hbm, t_hbm, f_hbm, outv_hbm, outf_hbm,
            cidx2, ridx2, rows_r, fr, accv, accf,
            semr, semc, semg, sems, semfg, semfs):
    c = lax.axis_index("c")
    s = lax.axis_index("s")

    def zbody(i, carry):
        for j in range(D // 16):
            rows_r[0, i, pl.ds(j * 16, 16)] = jnp.zeros((16,), jnp.float32)
        return carry

    lax.fori_loop(0, K, zbody, 0)
    for j in range(K // 16):
        fr[0, pl.ds(j * 16, 16)] = jnp.zeros((16,), jnp.float32)
    for j in range(SL // K):
        pltpu.sync_copy(rows_r.at[0], accv.at[pl.ds(s * SL + j * K, K)])
        pltpu.sync_copy(fr.at[0], accf.at[pl.ds(s * SL + j * K, K)])
    plsc.subcore_barrier()

    base = jnp.where(c == 0, s * BPW0, jnp.minimum(16 * BPW0 + s * BPW1,
                                                   EPAD // K - NB))
    nblk = jnp.where(c == 0, NBLK0, NBLK1)
    pltpu.async_copy(row_hbm.at[pl.ds(base, NB)], ridx2.at[0], semr[0]).wait()
    pltpu.async_copy(col_hbm.at[pl.ds(base, NB)], cidx2.at[0], semc[0]).wait()

    def one_block(noff, cur, nxt):
        ir = pltpu.async_copy(row_hbm.at[pl.ds(noff, NB)], ridx2.at[nxt],
                              semr[nxt])
        ic = pltpu.async_copy(col_hbm.at[pl.ds(noff, NB)], cidx2.at[nxt],
                              semc[nxt])
        ridx8 = ridx2.at[cur]
        cidx8 = cidx2.at[cur]
        gr = [None] * NRING
        sr = [None] * NRING
        gf = [None] * NRING
        sf = [None] * NRING
        for j in range(NB + LEAD):
            if j < NB:
                q = j % NRING
                if sr[q] is not None:
                    sr[q].wait()
                    sf[q].wait()
                gr[q] = pltpu.async_copy(t_hbm.at[cidx8.at[j]], rows_r.at[q],
                                         semg[q])
                gf[q] = pltpu.async_copy(f_hbm.at[cidx8.at[j]], fr.at[q],
                                         semfg[q])
            if j >= LEAD:
                p = (j - LEAD) % NRING
                gr[p].wait()
                gf[p].wait()
                sr[p] = pltpu.async_copy(rows_r.at[p],
                                         accv.at[ridx8.at[j - LEAD]],
                                         sems[p], add=True)
                sf[p] = pltpu.async_copy(fr.at[p],
                                         accf.at[ridx8.at[j - LEAD]],
                                         semfs[p], add=True)
        for q in range(NRING):
            if sr[q] is not None:
                sr[q].wait()
                sf[q].wait()
        ir.wait()
        ic.wait()

    def blk2(i, carry):
        b0 = i * 2
        one_block(base + ((b0 + 1) % nblk) * NB, 0, 1)
        one_block(base + ((b0 + 2) % nblk) * NB, 1, 0)
        return carry

    lax.fori_loop(0, nblk // 2, blk2, 0)
    plsc.subcore_barrier()
    pltpu.sync_copy(accv.at[pl.ds(s * SL, SL)], outv_hbm.at[c, pl.ds(s * SL, SL)])
    pltpu.sync_copy(accf.at[pl.ds(s * SL, SL)], outf_hbm.at[c, pl.ds(s * SL, SL)])


def _norm_kd(x):
    return jnp.clip(jnp.sqrt(jnp.sum(x * x, axis=-1, keepdims=True)),
                    _MIN_NORM, None)


def _proj(x, c):
    n = _norm_kd(x)
    maxnorm = (1.0 - _BALL_EPS) / jnp.sqrt(c)
    return jnp.where(n > maxnorm, x / n * maxnorm, x)


def _expmap0(u, c):
    sqrt_c = jnp.sqrt(c)
    u_norm = _norm_kd(u)
    return jnp.tanh(sqrt_c * u_norm) * u / (sqrt_c * u_norm)


def _mobius_add(x, y, c):
    x2 = jnp.sum(x * x, axis=-1, keepdims=True)
    y2 = jnp.sum(y * y, axis=-1, keepdims=True)
    xy = jnp.sum(x * y, axis=-1, keepdims=True)
    num = (1.0 + 2.0 * c * xy + c * y2) * x + (1.0 - c * x2) * y
    denom = 1.0 + 2.0 * c * xy + (c ** 2) * x2 * y2
    return num / jnp.clip(denom, _MIN_NORM, None)


def _p2k(x, c):
    return 2.0 * x / (1.0 + c * jnp.sum(x * x, axis=-1, keepdims=True))


def _k2p(x, c):
    denom = 1.0 + jnp.sqrt(
        jnp.clip(1.0 - c * jnp.sum(x * x, axis=-1, keepdims=True),
                 _MIN_NORM, None))
    return x / denom


def _dense_body(x_ref, w_ref, b_ref, d0_ref, d1_ref, c_ref, t_ref, f_ref):
    cc = c_ref[0, 0]
    xw = lax.dot_general(x_ref[...], w_ref[...], (((1,), (1,)), ((), ())),
                         preferred_element_type=jnp.float32)
    x_ = _proj(xw, cc)
    hb = _expmap0(b_ref[...], cc)
    x_ = _proj(_mobius_add(x_, hb, cc), cc)
    s = _p2k(x_, cc)
    lamb = lax.rsqrt(jnp.clip(1.0 - jnp.sum(s * s, axis=-1, keepdims=True),
                              _MIN_NORM, None))
    deg = 1.0 + d0_ref[...] + d1_ref[...]
    f = lax.rsqrt(deg) * lamb
    t_ref[...] = f * s
    f_ref[...] = f


def _fin_body(v_ref, s_ref, t_ref, f_ref, c_ref, o_ref):
    cc = c_ref[0, 0]
    num = v_ref[0] + v_ref[1] + t_ref[...]
    den = s_ref[0] + s_ref[1] + f_ref[...]
    out = _k2p(num / den, cc)
    o_ref[...] = jnp.where(out > 0, out, 0.01 * out)


_RB = 400


def _dense_call(x, W, bias, d0, d1, c_arr):
    return pl.pallas_call(
        _dense_body,
        grid=(N // _RB,),
        in_specs=[
            pl.BlockSpec((_RB, D), lambda i: (i, 0)),
            pl.BlockSpec((D, D), lambda i: (0, 0)),
            pl.BlockSpec((1, D), lambda i: (0, 0)),
            pl.BlockSpec((_RB, 1), lambda i: (i, 0)),
            pl.BlockSpec((_RB, 1), lambda i: (i, 0)),
            pl.BlockSpec((1, 1), lambda i: (0, 0)),
        ],
        out_specs=[
            pl.BlockSpec((_RB, D), lambda i: (i, 0)),
            pl.BlockSpec((_RB, 1), lambda i: (i, 0)),
        ],
        out_shape=[
            jax.ShapeDtypeStruct((N, D), jnp.float32),
            jax.ShapeDtypeStruct((N, 1), jnp.float32),
        ],
    )(x, W, bias, d0, d1, c_arr)


def _fin_call(aggv, aggs2, t, f, c_arr):
    return pl.pallas_call(
        _fin_body,
        grid=(N // _RB,),
        in_specs=[
            pl.BlockSpec((2, _RB, D), lambda i: (0, i, 0)),
            pl.BlockSpec((2, _RB, 1), lambda i: (0, i, 0)),
            pl.BlockSpec((_RB, D), lambda i: (i, 0)),
            pl.BlockSpec((_RB, 1), lambda i: (i, 0)),
            pl.BlockSpec((1, 1), lambda i: (0, 0)),
        ],
        out_specs=pl.BlockSpec((_RB, D), lambda i: (i, 0)),
        out_shape=jax.ShapeDtypeStruct((N, D), jnp.float32),
    )(aggv, aggs2, t, f, c_arr)


def kernel(x, edge_index, c_, W, bias):
    row = edge_index[0]
    col = edge_index[1]
    npd = EPAD - E
    pad_rows = N + jnp.arange(npd, dtype=jnp.int32) % (NPAD - N)
    rowp = jnp.concatenate([row, pad_rows])
    colp = jnp.concatenate([col, jnp.zeros((npd,), jnp.int32)])
    row3 = rowp.reshape(EPAD // K, K)
    col3 = colp.reshape(EPAD // K, K)

    degp = _deg_sc(row3)
    c_arr = jnp.full((1, 1), c_, jnp.float32)
    d0 = degp[0][:, None]
    d1 = degp[1][:, None]
    t, f = _dense_call(x, W, bias, d0, d1, c_arr)

    aggv, aggs = _agg_sc(row3, col3, t, f.reshape(N))
    aggs2 = aggs[:, :, None]
    return _fin_call(aggv, aggs2, t, f, c_arr)
